# baseline (speedup 1.0000x reference)
import functools

import jax
import jax.numpy as jnp
from jax import lax
from jax.experimental import pallas as pl
from jax.experimental.pallas import tpu as pltpu
from jax.experimental.pallas import tpu_sc as plsc

T = 2048
DIM = 2048
INTER = 1408
NE = 16
NSHARED = 2
SH_INTER = NSHARED * INTER

BLK = 256
NPAIR = 2 * T
NBA = NPAIR // BLK + NE
NB = NBA + 1
PAD = NB * BLK
TBLK = 256

NC = 2
NS = 16
NW = NC * NS
PPW = NPAIR // NW
CH = 32
NCH = PPW // CH


def _router_body(x_ref, gw_ref, topw_ref, dest_ref, be_ref, xq_ref):
    x = x_ref[...]
    xi = jax.lax.bitcast_convert_type(x.astype(jnp.bfloat16), jnp.int16)
    lo = xi[:, :DIM // 2].astype(jnp.int32) & 0xFFFF
    hi = xi[:, DIM // 2:].astype(jnp.int32) << 16
    xq_ref[...] = hi | lo
    gw = gw_ref[...]
    scores = jax.lax.dot_general(x, gw, (((1,), (1,)), ((), ())))
    s = jax.nn.softmax(scores, axis=-1)
    lane = lax.broadcasted_iota(jnp.int32, (T, NE), 1)
    m1 = jnp.max(s, axis=-1, keepdims=True)
    i1 = jnp.min(jnp.where(s == m1, lane, NE), axis=-1, keepdims=True)
    first = lane == i1
    s2 = jnp.where(first, -jnp.inf, s)
    m2 = jnp.max(s2, axis=-1, keepdims=True)
    i2 = jnp.min(jnp.where(s2 == m2, lane, NE), axis=-1, keepdims=True)
    second = lane == i2
    topw_ref[...] = jnp.concatenate([m1, m2], axis=1)

    oh0 = first.astype(jnp.float32)
    oh1 = second.astype(jnp.float32)
    r_iota = lax.broadcasted_iota(jnp.int32, (T, T), 0)
    c_iota = lax.broadcasted_iota(jnp.int32, (T, T), 1)
    stri = (r_iota > c_iota).astype(jnp.float32)
    oh01 = jnp.concatenate([oh0, oh1], axis=1)
    r01 = jax.lax.dot_general(stri.astype(jnp.bfloat16),
                              oh01.astype(jnp.bfloat16), (((1,), (0,)), ((), ())),
                              preferred_element_type=jnp.float32)
    r0 = r01[:, :NE]
    r1 = r01[:, NE:]
    c0 = jnp.sum(oh0, axis=0, keepdims=True)
    counts = c0 + jnp.sum(oh1, axis=0, keepdims=True)
    nb = jnp.floor((counts + (BLK - 1)) * (1.0 / BLK))
    e_r = lax.broadcasted_iota(jnp.int32, (NE, NE), 0)
    e_c = lax.broadcasted_iota(jnp.int32, (NE, NE), 1)
    incl = (e_r <= e_c).astype(jnp.float32)
    nb8 = jnp.broadcast_to(nb, (8, NE))
    cb = jax.lax.dot_general(nb8, incl, (((1,), (0,)), ((), ())),
                             preferred_element_type=jnp.float32)[0:1]
    base = (cb - nb) * float(BLK)
    base0 = jnp.sum(oh0 * base, axis=1, keepdims=True)
    base1 = jnp.sum(oh1 * base, axis=1, keepdims=True)
    rank0 = jnp.sum(oh0 * r0, axis=1, keepdims=True)
    rank1 = jnp.sum(oh1 * (r1 + c0), axis=1, keepdims=True)
    d0 = (base0 + rank0).astype(jnp.int32)
    d1 = (base1 + rank1).astype(jnp.int32)
    dest_ref[...] = jnp.concatenate([d0, d1], axis=1)

    lane16 = lax.broadcasted_iota(jnp.int32, (1, NE), 1)
    lastact = jnp.max(jnp.where(nb > 0, lane16, 0), axis=1, keepdims=True)
    cbb = jnp.broadcast_to(cb.astype(jnp.int32), (NB, NE))
    bio = lax.broadcasted_iota(jnp.int32, (NB, NE), 0)
    be = jnp.sum((cbb <= bio).astype(jnp.int32), axis=1, keepdims=True)
    nactive = cb.astype(jnp.int32)[0:1, NE - 1:NE]
    active = bio[:, 0:1] < nactive
    be = jnp.where(active, jnp.minimum(be, NE - 1), -1 - lastact)
    be_ref[...] = be


def _dispatch_body(x_hbm, destT_hbm, xd_hbm, idx_v, rows_v, sem):
    wid = lax.axis_index("s") * NC + lax.axis_index("c")
    k = wid // NS
    t0 = (wid % NS) * PPW
    for c4 in range(NCH):
        ts = t0 + c4 * CH
        pltpu.sync_copy(destT_hbm.at[k, pl.ds(ts, CH)], idx_v.at[c4])
        pltpu.sync_copy(x_hbm.at[pl.ds(ts, CH)], rows_v)
        pltpu.async_copy(rows_v, xd_hbm.at[idx_v.at[c4]], sem).wait()


def _gather_body(out2_hbm, destT_hbm, gath_hbm, idx_v, rows_v, sem):
    wid = lax.axis_index("s") * NC + lax.axis_index("c")
    k = wid // NS
    t0 = (wid % NS) * PPW
    pb = wid * PPW
    for c4 in range(NCH):
        ts = t0 + c4 * CH
        pltpu.sync_copy(destT_hbm.at[k, pl.ds(ts, CH)], idx_v.at[c4])
        pltpu.async_copy(out2_hbm.at[idx_v.at[c4]], rows_v, sem).wait()
        pltpu.sync_copy(rows_v, gath_hbm.at[pl.ds(pb + c4 * CH, CH)])


def _g1_body(be_ref, xd_ref, w1_ref, w3_ref, h_ref):
    @pl.when(be_ref[pl.program_id(0)] >= 0)
    def _():
        xdq = xd_ref[...]
        lo = jax.lax.bitcast_convert_type(xdq.astype(jnp.int16), jnp.bfloat16)
        hi = jax.lax.bitcast_convert_type((xdq >> 16).astype(jnp.int16),
                                          jnp.bfloat16)
        xb = jnp.concatenate([lo, hi], axis=1)
        dn = (((1,), (1,)), ((), ()))
        a = jax.lax.dot_general(xb, w1_ref[0, 0].astype(jnp.bfloat16), dn,
                                preferred_element_type=jnp.float32)
        b = jax.lax.dot_general(xb, w3_ref[0, 0].astype(jnp.bfloat16), dn,
                                preferred_element_type=jnp.float32)
        hv = jax.lax.bitcast_convert_type(
            (a * jax.nn.sigmoid(a) * b).astype(jnp.bfloat16), jnp.int16)
        Q = INTER // 4
        h_ref[0] = _pack_half(hv, (0, Q), (Q, 2 * Q))


def _pack_half(xi, lo_cols, hi_cols):
    lo = xi[:, lo_cols[0]:lo_cols[1]].astype(jnp.int32) & 0xFFFF
    hi = xi[:, hi_cols[0]:hi_cols[1]].astype(jnp.int32) << 16
    return hi | lo


def _g2_body(be_ref, h_ref, w2_ref, o_ref):
    @pl.when(be_ref[pl.program_id(0)] >= 0)
    def _():
        dn = (((1,), (1,)), ((), ()))
        hq = h_ref[...]
        pieces = []
        for ic in range(2):
            p = hq[ic]
            pieces.append(jax.lax.bitcast_convert_type(p.astype(jnp.int16),
                                                       jnp.bfloat16))
            pieces.append(jax.lax.bitcast_convert_type(
                (p >> 16).astype(jnp.int16), jnp.bfloat16))
        hb = jnp.concatenate(pieces, axis=1)
        out = jax.lax.dot_general(hb, w2_ref[0, 0].astype(jnp.bfloat16),
                                  dn, preferred_element_type=jnp.float32)
        xi = jax.lax.bitcast_convert_type(out.astype(jnp.bfloat16), jnp.int16)
        Q = DIM // 4
        dc = pl.program_id(1)
        o_ref[:, pl.ds(dc * Q, Q)] = _pack_half(xi, (0, Q), (Q, 2 * Q))


def _bidx(be, b):
    return jnp.where(be[b] >= 0, be[b], -1 - be[b])


def _rowidx(be, b):
    return jnp.where(be[b] >= 0, b, NB - 1)


def _sha_body(x_ref, ws1_ref, ws3_ref, hs_ref):
    xb = x_ref[...].astype(jnp.bfloat16)
    dn = (((1,), (1,)), ((), ()))
    a = jax.lax.dot_general(xb, ws1_ref[...].astype(jnp.bfloat16), dn,
                            preferred_element_type=jnp.float32)
    b = jax.lax.dot_general(xb, ws3_ref[...].astype(jnp.bfloat16), dn,
                            preferred_element_type=jnp.float32)
    hs_ref[...] = (a * jax.nn.sigmoid(a) * b).astype(jnp.bfloat16)


def _unpack(q):
    lo = jax.lax.bitcast_convert_type(q.astype(jnp.int16), jnp.bfloat16)
    hi = jax.lax.bitcast_convert_type((q >> 16).astype(jnp.int16),
                                      jnp.bfloat16)
    return jnp.concatenate([lo, hi], axis=1).astype(jnp.float32)


def _shb_body(hs_ref, ws2_ref, g0_ref, g1_ref, topw_ref, y_ref):
    dn = (((1,), (1,)), ((), ()))
    z = jax.lax.dot_general(hs_ref[...], ws2_ref[...].astype(jnp.bfloat16),
                            dn, preferred_element_type=jnp.float32)
    w0 = topw_ref[:, 0:1]
    w1 = topw_ref[:, 1:2]
    y_ref[...] = z + w0 * _unpack(g0_ref[...]) + w1 * _unpack(g1_ref[...])


def _dispatch():
    return pl.kernel(
        _dispatch_body,
        out_type=jax.ShapeDtypeStruct((PAD, DIM // 2), jnp.int32),
        mesh=plsc.VectorSubcoreMesh(core_axis_name="c", subcore_axis_name="s"),
        scratch_types=[
            pltpu.VMEM((NCH, CH), jnp.int32),
            pltpu.VMEM((CH, DIM // 2), jnp.int32),
            pltpu.SemaphoreType.DMA,
        ],
    )


def _gather():
    return pl.kernel(
        _gather_body,
        out_type=jax.ShapeDtypeStruct((NPAIR, DIM // 2), jnp.int32),
        mesh=plsc.VectorSubcoreMesh(core_axis_name="c", subcore_axis_name="s"),
        scratch_types=[
            pltpu.VMEM((NCH, CH), jnp.int32),
            pltpu.VMEM((CH, DIM // 2), jnp.int32),
            pltpu.SemaphoreType.DMA,
        ],
    )


def kernel(x, gate_w, w1, w2, w3, ws1, ws2, ws3, start_pos):
    del start_pos
    topw, dest, be, xq = pl.pallas_call(
        _router_body,
        out_shape=(
            jax.ShapeDtypeStruct((T, 2), jnp.float32),
            jax.ShapeDtypeStruct((T, 2), jnp.int32),
            jax.ShapeDtypeStruct((NB, 1), jnp.int32),
            jax.ShapeDtypeStruct((T, DIM // 2), jnp.int32),
        ),
    )(x, gate_w)
    destT = dest.T
    be1 = be.reshape((NB,))

    xd = _dispatch()(xq, destT)

    hs = pl.pallas_call(
        _sha_body,
        grid=(NSHARED, T // TBLK),
        in_specs=[
            pl.BlockSpec((TBLK, DIM), lambda ic, tb: (tb, 0)),
            pl.BlockSpec((INTER, DIM), lambda ic, tb: (ic, 0)),
            pl.BlockSpec((INTER, DIM), lambda ic, tb: (ic, 0)),
        ],
        out_specs=pl.BlockSpec((TBLK, INTER), lambda ic, tb: (tb, ic)),
        out_shape=jax.ShapeDtypeStruct((T, SH_INTER), jnp.bfloat16),
    )(x, ws1, ws3)

    w1r = w1.reshape(NE, 2, INTER // 2, DIM)
    w3r = w3.reshape(NE, 2, INTER // 2, DIM)
    h = pl.pallas_call(
        _g1_body,
        grid_spec=pltpu.PrefetchScalarGridSpec(
            num_scalar_prefetch=1,
            grid=(NB, 2),
            in_specs=[
                pl.BlockSpec((BLK, DIM // 2),
                             lambda b, ic, be: (_rowidx(be, b), 0)),
                pl.BlockSpec((1, 1, INTER // 2, DIM),
                             lambda b, ic, be: (_bidx(be, b), ic, 0, 0)),
                pl.BlockSpec((1, 1, INTER // 2, DIM),
                             lambda b, ic, be: (_bidx(be, b), ic, 0, 0)),
            ],
            out_specs=pl.BlockSpec((1, BLK, INTER // 4),
                                   lambda b, ic, be: (ic, _rowidx(be, b), 0)),
        ),
        out_shape=jax.ShapeDtypeStruct((2, PAD, INTER // 4), jnp.int32),
    )(be1, xd, w1r, w3r)

    w2r = w2.reshape(NE, 2, DIM // 2, INTER)
    out2 = pl.pallas_call(
        _g2_body,
        grid_spec=pltpu.PrefetchScalarGridSpec(
            num_scalar_prefetch=1,
            grid=(NB, 2),
            in_specs=[
                pl.BlockSpec((2, BLK, INTER // 4),
                             lambda b, dc, be: (0, _rowidx(be, b), 0)),
                pl.BlockSpec((1, 1, DIM // 2, INTER),
                             lambda b, dc, be: (_bidx(be, b), dc, 0, 0)),
            ],
            out_specs=pl.BlockSpec((BLK, DIM // 2),
                                   lambda b, dc, be: (_rowidx(be, b), 0)),
        ),
        out_shape=jax.ShapeDtypeStruct((PAD, DIM // 2), jnp.int32),
    )(be1, h, w2r)

    gath = _gather()(out2, destT)

    DC = 2
    y = pl.pallas_call(
        _shb_body,
        grid=(DC, T // TBLK),
        in_specs=[
            pl.BlockSpec((TBLK, SH_INTER), lambda dc, tb: (tb, 0)),
            pl.BlockSpec((DIM // DC, SH_INTER), lambda dc, tb: (dc, 0)),
            pl.BlockSpec((TBLK, DIM // DC // 2), lambda dc, tb: (tb, dc)),
            pl.BlockSpec((TBLK, DIM // DC // 2),
                         lambda dc, tb: (tb + T // TBLK, dc)),
            pl.BlockSpec((TBLK, 2), lambda dc, tb: (tb, 0)),
        ],
        out_specs=pl.BlockSpec((TBLK, DIM // DC), lambda dc, tb: (tb, dc)),
        out_shape=jax.ShapeDtypeStruct((T, DIM), jnp.float32),
    )(hs, ws2, gath, gath, topw)
    return y


# speedup vs baseline: 1.3437x; 1.3437x over previous
import functools

import jax
import jax.numpy as jnp
from jax import lax
from jax.experimental import pallas as pl
from jax.experimental.pallas import tpu as pltpu
from jax.experimental.pallas import tpu_sc as plsc

T = 2048
DIM = 2048
INTER = 1408
NE = 16
NSHARED = 2
SH_INTER = NSHARED * INTER

BLK = 256
NPAIR = 2 * T
NBA = NPAIR // BLK + NE
NB = NBA + 1
PAD = NB * BLK
TBLK = 256

NC = 2
NS = 16
NW = NC * NS
PPW = NPAIR // NW
CH = 32
NCH = PPW // CH


def _router_body(x_ref, gw_ref, topw_ref, dest_ref, be_ref, xq_ref):
    x = x_ref[...]
    xi = jax.lax.bitcast_convert_type(x.astype(jnp.bfloat16), jnp.int16)
    lo = xi[:, :DIM // 2].astype(jnp.int32) & 0xFFFF
    hi = xi[:, DIM // 2:].astype(jnp.int32) << 16
    xq_ref[...] = hi | lo
    gw = gw_ref[...]
    scores = jax.lax.dot_general(x, gw, (((1,), (1,)), ((), ())))
    s = jax.nn.softmax(scores, axis=-1)
    lane = lax.broadcasted_iota(jnp.int32, (T, NE), 1)
    m1 = jnp.max(s, axis=-1, keepdims=True)
    i1 = jnp.min(jnp.where(s == m1, lane, NE), axis=-1, keepdims=True)
    first = lane == i1
    s2 = jnp.where(first, -jnp.inf, s)
    m2 = jnp.max(s2, axis=-1, keepdims=True)
    i2 = jnp.min(jnp.where(s2 == m2, lane, NE), axis=-1, keepdims=True)
    second = lane == i2
    topw_ref[...] = jnp.concatenate([m1, m2], axis=1)

    oh0 = first.astype(jnp.float32)
    oh1 = second.astype(jnp.float32)
    r_iota = lax.broadcasted_iota(jnp.int32, (T, T), 0)
    c_iota = lax.broadcasted_iota(jnp.int32, (T, T), 1)
    stri = (r_iota > c_iota).astype(jnp.float32)
    oh01 = jnp.concatenate([oh0, oh1], axis=1)
    r01 = jax.lax.dot_general(stri.astype(jnp.bfloat16),
                              oh01.astype(jnp.bfloat16), (((1,), (0,)), ((), ())),
                              preferred_element_type=jnp.float32)
    r0 = r01[:, :NE]
    r1 = r01[:, NE:]
    c0 = jnp.sum(oh0, axis=0, keepdims=True)
    counts = c0 + jnp.sum(oh1, axis=0, keepdims=True)
    nb = jnp.floor((counts + (BLK - 1)) * (1.0 / BLK))
    e_r = lax.broadcasted_iota(jnp.int32, (NE, NE), 0)
    e_c = lax.broadcasted_iota(jnp.int32, (NE, NE), 1)
    incl = (e_r <= e_c).astype(jnp.float32)
    nb8 = jnp.broadcast_to(nb, (8, NE))
    cb = jax.lax.dot_general(nb8, incl, (((1,), (0,)), ((), ())),
                             preferred_element_type=jnp.float32)[0:1]
    base = (cb - nb) * float(BLK)
    base0 = jnp.sum(oh0 * base, axis=1, keepdims=True)
    base1 = jnp.sum(oh1 * base, axis=1, keepdims=True)
    rank0 = jnp.sum(oh0 * r0, axis=1, keepdims=True)
    rank1 = jnp.sum(oh1 * (r1 + c0), axis=1, keepdims=True)
    d0 = (base0 + rank0).astype(jnp.int32)
    d1 = (base1 + rank1).astype(jnp.int32)
    dest_ref[...] = jnp.concatenate([d0, d1], axis=1)

    lane16 = lax.broadcasted_iota(jnp.int32, (1, NE), 1)
    lastact = jnp.max(jnp.where(nb > 0, lane16, 0), axis=1, keepdims=True)
    cbb = jnp.broadcast_to(cb.astype(jnp.int32), (NB, NE))
    bio = lax.broadcasted_iota(jnp.int32, (NB, NE), 0)
    be = jnp.sum((cbb <= bio).astype(jnp.int32), axis=1, keepdims=True)
    nactive = cb.astype(jnp.int32)[0:1, NE - 1:NE]
    active = bio[:, 0:1] < nactive
    be = jnp.where(active, jnp.minimum(be, NE - 1), -1 - lastact)
    be_ref[...] = be


def _dispatch_body(x_hbm, destT_hbm, xd_hbm, idx_v, rows_v, sem):
    wid = lax.axis_index("s") * NC + lax.axis_index("c")
    k = wid // NS
    t0 = (wid % NS) * PPW
    for c4 in range(NCH):
        ts = t0 + c4 * CH
        pltpu.sync_copy(destT_hbm.at[k, pl.ds(ts, CH)], idx_v.at[c4])
        pltpu.sync_copy(x_hbm.at[pl.ds(ts, CH)], rows_v)
        pltpu.async_copy(rows_v, xd_hbm.at[idx_v.at[c4]], sem).wait()


def _gather_body(out2_hbm, destT_hbm, gath_hbm, idx_v, rows_v, sem):
    wid = lax.axis_index("s") * NC + lax.axis_index("c")
    k = wid // NS
    t0 = (wid % NS) * PPW
    pb = wid * PPW
    for c4 in range(NCH):
        ts = t0 + c4 * CH
        pltpu.sync_copy(destT_hbm.at[k, pl.ds(ts, CH)], idx_v.at[c4])
        pltpu.async_copy(out2_hbm.at[idx_v.at[c4]], rows_v, sem).wait()
        pltpu.sync_copy(rows_v, gath_hbm.at[pl.ds(pb + c4 * CH, CH)])


def _g1_body(be_ref, xd_ref, w1_ref, w3_ref, h_ref):
    @pl.when(be_ref[pl.program_id(0)] >= 0)
    def _():
        xdq = xd_ref[...]
        lo = jax.lax.bitcast_convert_type(xdq.astype(jnp.int16), jnp.bfloat16)
        hi = jax.lax.bitcast_convert_type((xdq >> 16).astype(jnp.int16),
                                          jnp.bfloat16)
        xb = jnp.concatenate([lo, hi], axis=1)
        dn = (((1,), (1,)), ((), ()))
        a = jax.lax.dot_general(xb, w1_ref[0].astype(jnp.bfloat16), dn,
                                preferred_element_type=jnp.float32)
        b = jax.lax.dot_general(xb, w3_ref[0].astype(jnp.bfloat16), dn,
                                preferred_element_type=jnp.float32)
        hv = jax.lax.bitcast_convert_type(
            (a * jax.nn.sigmoid(a) * b).astype(jnp.bfloat16), jnp.int16)
        Q = INTER // 2
        h_ref[...] = _pack_half(hv, (0, Q), (Q, 2 * Q))


def _pack_half(xi, lo_cols, hi_cols):
    lo = xi[:, lo_cols[0]:lo_cols[1]].astype(jnp.int32) & 0xFFFF
    hi = xi[:, hi_cols[0]:hi_cols[1]].astype(jnp.int32) << 16
    return hi | lo


def _g2_body(be_ref, h_ref, w2_ref, o_ref):
    @pl.when(be_ref[pl.program_id(0)] >= 0)
    def _():
        dn = (((1,), (1,)), ((), ()))
        hq = h_ref[...]
        hlo = jax.lax.bitcast_convert_type(hq.astype(jnp.int16), jnp.bfloat16)
        hhi = jax.lax.bitcast_convert_type((hq >> 16).astype(jnp.int16),
                                           jnp.bfloat16)
        hb = jnp.concatenate([hlo, hhi], axis=1)
        out = jax.lax.dot_general(hb, w2_ref[0].astype(jnp.bfloat16),
                                  dn, preferred_element_type=jnp.float32)
        xi = jax.lax.bitcast_convert_type(out.astype(jnp.bfloat16), jnp.int16)
        Q = DIM // 4
        ql = _pack_half(xi, (0, Q), (Q, 2 * Q))
        qr = _pack_half(xi, (2 * Q, 3 * Q), (3 * Q, 4 * Q))
        o_ref[...] = jnp.concatenate([ql, qr], axis=1)


def _bidx(be, b):
    return jnp.where(be[b] >= 0, be[b], -1 - be[b])


def _rowidx(be, b):
    return jnp.where(be[b] >= 0, b, NB - 1)


def _sha_body(x_ref, ws1_ref, ws3_ref, hs_ref):
    xb = x_ref[...].astype(jnp.bfloat16)
    dn = (((1,), (1,)), ((), ()))
    a = jax.lax.dot_general(xb, ws1_ref[...].astype(jnp.bfloat16), dn,
                            preferred_element_type=jnp.float32)
    b = jax.lax.dot_general(xb, ws3_ref[...].astype(jnp.bfloat16), dn,
                            preferred_element_type=jnp.float32)
    hs_ref[...] = (a * jax.nn.sigmoid(a) * b).astype(jnp.bfloat16)


def _unpack(q):
    lo = jax.lax.bitcast_convert_type(q.astype(jnp.int16), jnp.bfloat16)
    hi = jax.lax.bitcast_convert_type((q >> 16).astype(jnp.int16),
                                      jnp.bfloat16)
    return jnp.concatenate([lo, hi], axis=1).astype(jnp.float32)


def _shb_body(hs_ref, ws2_ref, g0_ref, g1_ref, topw_ref, y_ref):
    dn = (((1,), (1,)), ((), ()))
    z = jax.lax.dot_general(hs_ref[...], ws2_ref[...].astype(jnp.bfloat16),
                            dn, preferred_element_type=jnp.float32)
    w0 = topw_ref[:, 0:1]
    w1 = topw_ref[:, 1:2]
    y_ref[...] = z + w0 * _unpack(g0_ref[...]) + w1 * _unpack(g1_ref[...])


def _dispatch():
    return pl.kernel(
        _dispatch_body,
        out_type=jax.ShapeDtypeStruct((PAD, DIM // 2), jnp.int32),
        mesh=plsc.VectorSubcoreMesh(core_axis_name="c", subcore_axis_name="s"),
        scratch_types=[
            pltpu.VMEM((NCH, CH), jnp.int32),
            pltpu.VMEM((CH, DIM // 2), jnp.int32),
            pltpu.SemaphoreType.DMA,
        ],
    )


def _gather():
    return pl.kernel(
        _gather_body,
        out_type=jax.ShapeDtypeStruct((NPAIR, DIM // 2), jnp.int32),
        mesh=plsc.VectorSubcoreMesh(core_axis_name="c", subcore_axis_name="s"),
        scratch_types=[
            pltpu.VMEM((NCH, CH), jnp.int32),
            pltpu.VMEM((CH, DIM // 2), jnp.int32),
            pltpu.SemaphoreType.DMA,
        ],
    )


def kernel(x, gate_w, w1, w2, w3, ws1, ws2, ws3, start_pos):
    del start_pos
    topw, dest, be, xq = pl.pallas_call(
        _router_body,
        out_shape=(
            jax.ShapeDtypeStruct((T, 2), jnp.float32),
            jax.ShapeDtypeStruct((T, 2), jnp.int32),
            jax.ShapeDtypeStruct((NB, 1), jnp.int32),
            jax.ShapeDtypeStruct((T, DIM // 2), jnp.int32),
        ),
    )(x, gate_w)
    destT = dest.T
    be1 = be.reshape((NB,))

    xd = _dispatch()(xq, destT)

    hs = pl.pallas_call(
        _sha_body,
        grid=(NSHARED, T // TBLK),
        in_specs=[
            pl.BlockSpec((TBLK, DIM), lambda ic, tb: (tb, 0)),
            pl.BlockSpec((INTER, DIM), lambda ic, tb: (ic, 0)),
            pl.BlockSpec((INTER, DIM), lambda ic, tb: (ic, 0)),
        ],
        out_specs=pl.BlockSpec((TBLK, INTER), lambda ic, tb: (tb, ic)),
        out_shape=jax.ShapeDtypeStruct((T, SH_INTER), jnp.bfloat16),
    )(x, ws1, ws3)

    h = pl.pallas_call(
        _g1_body,
        grid_spec=pltpu.PrefetchScalarGridSpec(
            num_scalar_prefetch=1,
            grid=(NB,),
            in_specs=[
                pl.BlockSpec((BLK, DIM // 2),
                             lambda b, be: (_rowidx(be, b), 0)),
                pl.BlockSpec((1, INTER, DIM),
                             lambda b, be: (_bidx(be, b), 0, 0)),
                pl.BlockSpec((1, INTER, DIM),
                             lambda b, be: (_bidx(be, b), 0, 0)),
            ],
            out_specs=pl.BlockSpec((BLK, INTER // 2),
                                   lambda b, be: (_rowidx(be, b), 0)),
        ),
        out_shape=jax.ShapeDtypeStruct((PAD, INTER // 2), jnp.int32),
    )(be1, xd, w1, w3)

    out2 = pl.pallas_call(
        _g2_body,
        grid_spec=pltpu.PrefetchScalarGridSpec(
            num_scalar_prefetch=1,
            grid=(NB,),
            in_specs=[
                pl.BlockSpec((BLK, INTER // 2),
                             lambda b, be: (_rowidx(be, b), 0)),
                pl.BlockSpec((1, DIM, INTER),
                             lambda b, be: (_bidx(be, b), 0, 0)),
            ],
            out_specs=pl.BlockSpec((BLK, DIM // 2),
                                   lambda b, be: (_rowidx(be, b), 0)),
        ),
        out_shape=jax.ShapeDtypeStruct((PAD, DIM // 2), jnp.int32),
    )(be1, h, w2)

    gath = _gather()(out2, destT)

    DC = 2
    y = pl.pallas_call(
        _shb_body,
        grid=(DC, T // TBLK),
        in_specs=[
            pl.BlockSpec((TBLK, SH_INTER), lambda dc, tb: (tb, 0)),
            pl.BlockSpec((DIM // DC, SH_INTER), lambda dc, tb: (dc, 0)),
            pl.BlockSpec((TBLK, DIM // DC // 2), lambda dc, tb: (tb, dc)),
            pl.BlockSpec((TBLK, DIM // DC // 2),
                         lambda dc, tb: (tb + T // TBLK, dc)),
            pl.BlockSpec((TBLK, 2), lambda dc, tb: (tb, 0)),
        ],
        out_specs=pl.BlockSpec((TBLK, DIM // DC), lambda dc, tb: (tb, dc)),
        out_shape=jax.ShapeDtypeStruct((T, DIM), jnp.float32),
    )(hs, ws2, gath, gath, topw)
    return y


# speedup vs baseline: 1.3587x; 1.0111x over previous
import functools

import jax
import jax.numpy as jnp
from jax import lax
from jax.experimental import pallas as pl
from jax.experimental.pallas import tpu as pltpu
from jax.experimental.pallas import tpu_sc as plsc

T = 2048
DIM = 2048
INTER = 1408
NE = 16
NSHARED = 2
SH_INTER = NSHARED * INTER

BLK = 256
NPAIR = 2 * T
NBA = NPAIR // BLK + NE
NB = NBA + 1
PAD = NB * BLK
TBLK = 256

NC = 2
NS = 16
NW = NC * NS
PPW = NPAIR // NW
CH = 32
NCH = PPW // CH


def _router_body(x_ref, gw_ref, topw_ref, dest_ref, be_ref, xq_ref):
    x = x_ref[...]
    xi = jax.lax.bitcast_convert_type(x.astype(jnp.bfloat16), jnp.int16)
    lo = xi[:, :DIM // 2].astype(jnp.int32) & 0xFFFF
    hi = xi[:, DIM // 2:].astype(jnp.int32) << 16
    xq_ref[...] = hi | lo
    gw = gw_ref[...]
    scores = jax.lax.dot_general(x, gw, (((1,), (1,)), ((), ())))
    s = jax.nn.softmax(scores, axis=-1)
    lane = lax.broadcasted_iota(jnp.int32, (T, NE), 1)
    m1 = jnp.max(s, axis=-1, keepdims=True)
    i1 = jnp.min(jnp.where(s == m1, lane, NE), axis=-1, keepdims=True)
    first = lane == i1
    s2 = jnp.where(first, -jnp.inf, s)
    m2 = jnp.max(s2, axis=-1, keepdims=True)
    i2 = jnp.min(jnp.where(s2 == m2, lane, NE), axis=-1, keepdims=True)
    second = lane == i2
    topw_ref[...] = jnp.concatenate([m1, m2], axis=1)

    oh0 = first.astype(jnp.float32)
    oh1 = second.astype(jnp.float32)
    r_iota = lax.broadcasted_iota(jnp.int32, (T, T), 0)
    c_iota = lax.broadcasted_iota(jnp.int32, (T, T), 1)
    stri = (r_iota > c_iota).astype(jnp.float32)
    oh01 = jnp.concatenate([oh0, oh1], axis=1)
    r01 = jax.lax.dot_general(stri.astype(jnp.bfloat16),
                              oh01.astype(jnp.bfloat16), (((1,), (0,)), ((), ())),
                              preferred_element_type=jnp.float32)
    r0 = r01[:, :NE]
    r1 = r01[:, NE:]
    c0 = jnp.sum(oh0, axis=0, keepdims=True)
    counts = c0 + jnp.sum(oh1, axis=0, keepdims=True)
    nb = jnp.floor((counts + (BLK - 1)) * (1.0 / BLK))
    e_r = lax.broadcasted_iota(jnp.int32, (NE, NE), 0)
    e_c = lax.broadcasted_iota(jnp.int32, (NE, NE), 1)
    incl = (e_r <= e_c).astype(jnp.float32)
    nb8 = jnp.broadcast_to(nb, (8, NE))
    cb = jax.lax.dot_general(nb8, incl, (((1,), (0,)), ((), ())),
                             preferred_element_type=jnp.float32)[0:1]
    base = (cb - nb) * float(BLK)
    base0 = jnp.sum(oh0 * base, axis=1, keepdims=True)
    base1 = jnp.sum(oh1 * base, axis=1, keepdims=True)
    rank0 = jnp.sum(oh0 * r0, axis=1, keepdims=True)
    rank1 = jnp.sum(oh1 * (r1 + c0), axis=1, keepdims=True)
    d0 = (base0 + rank0).astype(jnp.int32)
    d1 = (base1 + rank1).astype(jnp.int32)
    dest_ref[...] = jnp.concatenate([d0, d1], axis=1)

    lane16 = lax.broadcasted_iota(jnp.int32, (1, NE), 1)
    lastact = jnp.max(jnp.where(nb > 0, lane16, 0), axis=1, keepdims=True)
    cbb = jnp.broadcast_to(cb.astype(jnp.int32), (NB, NE))
    bio = lax.broadcasted_iota(jnp.int32, (NB, NE), 0)
    be = jnp.sum((cbb <= bio).astype(jnp.int32), axis=1, keepdims=True)
    nactive = cb.astype(jnp.int32)[0:1, NE - 1:NE]
    active = bio[:, 0:1] < nactive
    be = jnp.where(active, jnp.minimum(be, NE - 1), -1 - lastact)
    be_ref[...] = be


def _dispatch_body(x_hbm, destT_hbm, xd_hbm, idx_v, rows_v, sem):
    wid = lax.axis_index("s") * NC + lax.axis_index("c")
    k = wid // NS
    t0 = (wid % NS) * PPW
    for c4 in range(NCH):
        ts = t0 + c4 * CH
        pltpu.sync_copy(destT_hbm.at[k, pl.ds(ts, CH)], idx_v.at[c4])
        pltpu.sync_copy(x_hbm.at[pl.ds(ts, CH)], rows_v)
        pltpu.async_copy(rows_v, xd_hbm.at[idx_v.at[c4]], sem).wait()


def _gather_body(out2_hbm, destT_hbm, gath_hbm, idx_v, rows_v, sem):
    wid = lax.axis_index("s") * NC + lax.axis_index("c")
    k = wid // NS
    t0 = (wid % NS) * PPW
    pb = wid * PPW
    for c4 in range(NCH):
        ts = t0 + c4 * CH
        pltpu.sync_copy(destT_hbm.at[k, pl.ds(ts, CH)], idx_v.at[c4])
        pltpu.async_copy(out2_hbm.at[idx_v.at[c4]], rows_v, sem).wait()
        pltpu.sync_copy(rows_v, gath_hbm.at[pl.ds(pb + c4 * CH, CH)])


def _g1_body(be_ref, xd_ref, w1_ref, w3_ref, h_ref):
    @pl.when(be_ref[pl.program_id(0)] >= 0)
    def _():
        xdq = xd_ref[...]
        lo = jax.lax.bitcast_convert_type(xdq.astype(jnp.int16), jnp.bfloat16)
        hi = jax.lax.bitcast_convert_type((xdq >> 16).astype(jnp.int16),
                                          jnp.bfloat16)
        xb = jnp.concatenate([lo, hi], axis=1)
        dn = (((1,), (1,)), ((), ()))
        a = jax.lax.dot_general(xb, w1_ref[0].astype(jnp.bfloat16), dn,
                                preferred_element_type=jnp.float32)
        b = jax.lax.dot_general(xb, w3_ref[0].astype(jnp.bfloat16), dn,
                                preferred_element_type=jnp.float32)
        hv = jax.lax.bitcast_convert_type(
            (a * jax.nn.sigmoid(a) * b).astype(jnp.bfloat16), jnp.int16)
        Q = INTER // 2
        h_ref[...] = _pack_half(hv, (0, Q), (Q, 2 * Q))


def _pack_half(xi, lo_cols, hi_cols):
    lo = xi[:, lo_cols[0]:lo_cols[1]].astype(jnp.int32) & 0xFFFF
    hi = xi[:, hi_cols[0]:hi_cols[1]].astype(jnp.int32) << 16
    return hi | lo


def _g2_body(be_ref, h_ref, w2_ref, o_ref):
    @pl.when(be_ref[pl.program_id(0)] >= 0)
    def _():
        dn = (((1,), (1,)), ((), ()))
        hq = h_ref[...]
        hlo = jax.lax.bitcast_convert_type(hq.astype(jnp.int16), jnp.bfloat16)
        hhi = jax.lax.bitcast_convert_type((hq >> 16).astype(jnp.int16),
                                           jnp.bfloat16)
        hb = jnp.concatenate([hlo, hhi], axis=1)
        out = jax.lax.dot_general(hb, w2_ref[0].astype(jnp.bfloat16),
                                  dn, preferred_element_type=jnp.float32)
        xi = jax.lax.bitcast_convert_type(out.astype(jnp.bfloat16), jnp.int16)
        Q = DIM // 4
        ql = _pack_half(xi, (0, Q), (Q, 2 * Q))
        qr = _pack_half(xi, (2 * Q, 3 * Q), (3 * Q, 4 * Q))
        o_ref[...] = jnp.concatenate([ql, qr], axis=1)


def _bidx(be, b):
    return jnp.where(be[b] >= 0, be[b], -1 - be[b])


def _rowidx(be, b):
    return jnp.where(be[b] >= 0, b, NB - 1)


def _sha_body(x_ref, ws1_ref, ws3_ref, hs_ref):
    xb = x_ref[...].astype(jnp.bfloat16)
    dn = (((1,), (1,)), ((), ()))
    a = jax.lax.dot_general(xb, ws1_ref[...].astype(jnp.bfloat16), dn,
                            preferred_element_type=jnp.float32)
    b = jax.lax.dot_general(xb, ws3_ref[...].astype(jnp.bfloat16), dn,
                            preferred_element_type=jnp.float32)
    hs_ref[...] = (a * jax.nn.sigmoid(a) * b).astype(jnp.bfloat16)


def _unpack(q):
    lo = jax.lax.bitcast_convert_type(q.astype(jnp.int16), jnp.bfloat16)
    hi = jax.lax.bitcast_convert_type((q >> 16).astype(jnp.int16),
                                      jnp.bfloat16)
    return jnp.concatenate([lo, hi], axis=1).astype(jnp.float32)


def _shb_body(hs_ref, ws2_ref, g0_ref, g1_ref, topw_ref, y_ref):
    dn = (((1,), (1,)), ((), ()))
    z = jax.lax.dot_general(hs_ref[...], ws2_ref[...].astype(jnp.bfloat16),
                            dn, preferred_element_type=jnp.float32)
    w0 = topw_ref[:, 0:1]
    w1 = topw_ref[:, 1:2]
    y_ref[...] = z + w0 * _unpack(g0_ref[...]) + w1 * _unpack(g1_ref[...])


def _dispatch():
    return pl.kernel(
        _dispatch_body,
        out_type=jax.ShapeDtypeStruct((PAD, DIM // 2), jnp.int32),
        mesh=plsc.VectorSubcoreMesh(core_axis_name="c", subcore_axis_name="s"),
        scratch_types=[
            pltpu.VMEM((NCH, CH), jnp.int32),
            pltpu.VMEM((CH, DIM // 2), jnp.int32),
            pltpu.SemaphoreType.DMA,
        ],
    )


def _gather():
    return pl.kernel(
        _gather_body,
        out_type=jax.ShapeDtypeStruct((NPAIR, DIM // 2), jnp.int32),
        mesh=plsc.VectorSubcoreMesh(core_axis_name="c", subcore_axis_name="s"),
        scratch_types=[
            pltpu.VMEM((NCH, CH), jnp.int32),
            pltpu.VMEM((CH, DIM // 2), jnp.int32),
            pltpu.SemaphoreType.DMA,
        ],
    )


def kernel(x, gate_w, w1, w2, w3, ws1, ws2, ws3, start_pos):
    del start_pos
    topw, dest, be, xq = pl.pallas_call(
        _router_body,
        out_shape=(
            jax.ShapeDtypeStruct((T, 2), jnp.float32),
            jax.ShapeDtypeStruct((T, 2), jnp.int32),
            jax.ShapeDtypeStruct((NB, 1), jnp.int32),
            jax.ShapeDtypeStruct((T, DIM // 2), jnp.int32),
        ),
    )(x, gate_w)
    destT = dest.T
    be1 = be.reshape((NB,))

    xd = _dispatch()(xq, destT)

    hs = pl.pallas_call(
        _sha_body,
        grid=(NSHARED, T // TBLK),
        in_specs=[
            pl.BlockSpec((TBLK, DIM), lambda ic, tb: (tb, 0)),
            pl.BlockSpec((INTER, DIM), lambda ic, tb: (ic, 0)),
            pl.BlockSpec((INTER, DIM), lambda ic, tb: (ic, 0)),
        ],
        out_specs=pl.BlockSpec((TBLK, INTER), lambda ic, tb: (tb, ic)),
        out_shape=jax.ShapeDtypeStruct((T, SH_INTER), jnp.bfloat16),
    )(x, ws1, ws3)

    h = pl.pallas_call(
        _g1_body,
        grid_spec=pltpu.PrefetchScalarGridSpec(
            num_scalar_prefetch=1,
            grid=(NB,),
            in_specs=[
                pl.BlockSpec((BLK, DIM // 2),
                             lambda b, be: (_rowidx(be, b), 0)),
                pl.BlockSpec((1, INTER, DIM),
                             lambda b, be: (_bidx(be, b), 0, 0)),
                pl.BlockSpec((1, INTER, DIM),
                             lambda b, be: (_bidx(be, b), 0, 0)),
            ],
            out_specs=pl.BlockSpec((BLK, INTER // 2),
                                   lambda b, be: (_rowidx(be, b), 0)),
        ),
        out_shape=jax.ShapeDtypeStruct((PAD, INTER // 2), jnp.int32),
    )(be1, xd, w1, w3)

    out2 = pl.pallas_call(
        _g2_body,
        grid_spec=pltpu.PrefetchScalarGridSpec(
            num_scalar_prefetch=1,
            grid=(NB,),
            in_specs=[
                pl.BlockSpec((BLK, INTER // 2),
                             lambda b, be: (_rowidx(be, b), 0)),
                pl.BlockSpec((1, DIM, INTER),
                             lambda b, be: (_bidx(be, b), 0, 0)),
            ],
            out_specs=pl.BlockSpec((BLK, DIM // 2),
                                   lambda b, be: (_rowidx(be, b), 0)),
        ),
        out_shape=jax.ShapeDtypeStruct((PAD, DIM // 2), jnp.int32),
    )(be1, h, w2)

    gath = _gather()(out2, destT)

    DC = 2
    TB2 = 512
    y = pl.pallas_call(
        _shb_body,
        grid=(DC, T // TB2),
        in_specs=[
            pl.BlockSpec((TB2, SH_INTER), lambda dc, tb: (tb, 0)),
            pl.BlockSpec((DIM // DC, SH_INTER), lambda dc, tb: (dc, 0)),
            pl.BlockSpec((TB2, DIM // DC // 2), lambda dc, tb: (tb, dc)),
            pl.BlockSpec((TB2, DIM // DC // 2),
                         lambda dc, tb: (tb + T // TB2, dc)),
            pl.BlockSpec((TB2, 2), lambda dc, tb: (tb, 0)),
        ],
        out_specs=pl.BlockSpec((TB2, DIM // DC), lambda dc, tb: (tb, dc)),
        out_shape=jax.ShapeDtypeStruct((T, DIM), jnp.float32),
    )(hs, ws2, gath, gath, topw)
    return y
